# baseline (speedup 1.0000x reference)
import functools

import jax
import jax.numpy as jnp
from jax import lax
from jax.experimental import pallas as pl
from jax.experimental.pallas import tpu as pltpu
from jax.experimental.pallas import tpu_sc as plsc

NC = 2
NS = 16
NW = NC * NS

BATCH = 4096
SEQ = 200
D = 64
TOTAL_ROWS = BATCH * SEQ
ROWS_PER_W = TOTAL_ROWS // NW
CH = 128
NCH = ROWS_PER_W // CH
NPAIR = NCH // 2


def _body(x_hbm, table_hbm, pe_hbm, out_hbm,
          idx_v, pe_v, rows_a, rows_b, sem_a, sem_b):
    cid = lax.axis_index("c")
    sid = lax.axis_index("s")
    wid = sid * NC + cid
    base = wid * ROWS_PER_W

    pltpu.sync_copy(x_hbm.at[pl.ds(base, ROWS_PER_W)], idx_v)
    pltpu.sync_copy(pe_hbm, pe_v)

    bufs = (rows_a, rows_b)
    sems = (sem_a, sem_b)

    def start_gather(j, buf, sem):
        return pltpu.async_copy(table_hbm.at[idx_v.at[pl.ds(j * CH, CH)]], buf, sem)

    def wait_gather(j, buf, sem):
        pltpu.make_async_copy(table_hbm.at[idx_v.at[pl.ds(j * CH, CH)]], buf, sem).wait()

    def process(j, buf):
        phase = lax.rem(j * CH, SEQ)

        def row(r, c2):
            off = phase + r
            off = jnp.where(off >= SEQ, off - SEQ, off)
            for c in range(D // 16):
                sl = pl.ds(c * 16, 16)
                plsc.addupdate(buf.at[r, sl], pe_v[off, sl])
            return c2

        lax.fori_loop(0, CH, row, 0, unroll=4)
        pltpu.sync_copy(buf, out_hbm.at[pl.ds(base + j * CH, CH)])

    start_gather(0, bufs[0], sems[0])

    def pair(p, carry):
        for parity in range(2):
            j = 2 * p + parity
            buf, sem = bufs[parity], sems[parity]
            nbuf, nsem = bufs[1 - parity], sems[1 - parity]
            wait_gather(j, buf, sem)
            @pl.when(j + 1 < NCH)
            def _():
                start_gather(j + 1, nbuf, nsem)
            process(j, buf)
        return carry

    lax.fori_loop(0, NPAIR, pair, 0)


def kernel(x, table, pe):
    x_flat = x.reshape(TOTAL_ROWS).astype(jnp.int32)
    pe2d = pe.reshape(SEQ, D).astype(jnp.float32)

    mesh = plsc.VectorSubcoreMesh(core_axis_name="c", subcore_axis_name="s")
    out = pl.kernel(
        _body,
        out_type=jax.ShapeDtypeStruct((TOTAL_ROWS, D), jnp.float32),
        mesh=mesh,
        scratch_types=[
            pltpu.VMEM((ROWS_PER_W,), jnp.int32),
            pltpu.VMEM((SEQ, D), jnp.float32),
            pltpu.VMEM((CH, D), jnp.float32),
            pltpu.VMEM((CH, D), jnp.float32),
            pltpu.SemaphoreType.DMA,
            pltpu.SemaphoreType.DMA,
        ],
        compiler_params=pltpu.CompilerParams(use_tc_tiling_on_sc=False),
    )(x_flat, table, pe2d)
    return out.reshape(BATCH, SEQ, D)
